# baseline (device time: 199663 ns/iter reference)
import jax
import jax.numpy as jnp
from jax import lax
from jax.experimental import pallas as pl
from jax.experimental.pallas import tpu as pltpu

N_DEV = 8


def kernel(x, w_mat, scale_x, scale_w):
    m_per, k = x.shape
    _, n_per = w_mat.shape

    x8 = x.astype(jnp.float8_e4m3fn)
    w16 = w_mat.astype(jnp.bfloat16)
    s = (scale_x * scale_w).reshape(1, 1)

    def body(s_ref, x_ref, w_ref, out_ref, comm_ref, send_sems, recv_sems):
        my = lax.axis_index("i")
        left = lax.rem(my + N_DEV - 1, N_DEV)
        right = lax.rem(my + 1, N_DEV)

        barrier_sem = pltpu.get_barrier_semaphore()
        for nbr in (left, right):
            pl.semaphore_signal(
                barrier_sem, inc=1,
                device_id=(nbr,), device_id_type=pl.DeviceIdType.MESH,
            )
        pl.semaphore_wait(barrier_sem, 2)

        sc = s_ref[0, 0]
        w = w_ref[...]

        out_ref[pl.ds(my * m_per, m_per), :] = (
            jnp.dot(x_ref[...].astype(jnp.bfloat16), w,
                    preferred_element_type=jnp.float32) * sc
        )

        for h in range(N_DEV - 1):
            src = x_ref if h == 0 else comm_ref.at[h - 1]
            rdma = pltpu.make_async_remote_copy(
                src_ref=src,
                dst_ref=comm_ref.at[h],
                send_sem=send_sems.at[h],
                recv_sem=recv_sems.at[h],
                device_id=(right,),
                device_id_type=pl.DeviceIdType.MESH,
            )
            rdma.start()
            rdma.wait()
            origin = lax.rem(my + N_DEV - 1 - h, N_DEV)
            out_ref[pl.ds(origin * m_per, m_per), :] = (
                jnp.dot(comm_ref[h].astype(jnp.bfloat16), w,
                        preferred_element_type=jnp.float32) * sc
            )

    return pl.pallas_call(
        body,
        out_shape=jax.ShapeDtypeStruct((N_DEV * m_per, n_per), jnp.float32),
        in_specs=[
            pl.BlockSpec(memory_space=pltpu.SMEM),
            pl.BlockSpec(memory_space=pltpu.VMEM),
            pl.BlockSpec(memory_space=pltpu.VMEM),
        ],
        out_specs=pl.BlockSpec(memory_space=pltpu.VMEM),
        scratch_shapes=[
            pltpu.VMEM((N_DEV - 1, m_per, k), jnp.float8_e4m3fn),
            pltpu.SemaphoreType.DMA((N_DEV - 1,)),
            pltpu.SemaphoreType.DMA((N_DEV - 1,)),
        ],
        compiler_params=pltpu.CompilerParams(collective_id=0),
    )(s, x8, w16)


# device time: 106209 ns/iter; 1.8799x vs baseline; 1.8799x over previous
import jax
import jax.numpy as jnp
from jax import lax
from jax.experimental import pallas as pl
from jax.experimental.pallas import tpu as pltpu

N_DEV = 8
N_HOP = N_DEV - 1


def kernel(x, w_mat, scale_x, scale_w):
    m_per, k = x.shape
    half = m_per // 2
    _, n_per = w_mat.shape

    x8 = x.astype(jnp.float8_e4m3fn)
    w16 = w_mat.astype(jnp.bfloat16)
    s = (scale_x * scale_w).reshape(1, 1)

    def body(s_ref, x_ref, w_ref, out_ref,
             cw_ref, ccw_ref, send_cw, recv_cw, send_ccw, recv_ccw):
        my = lax.axis_index("i")
        left = lax.rem(my + N_DEV - 1, N_DEV)
        right = lax.rem(my + 1, N_DEV)

        barrier_sem = pltpu.get_barrier_semaphore()
        for nbr in (left, right):
            pl.semaphore_signal(
                barrier_sem, inc=1,
                device_id=(nbr,), device_id_type=pl.DeviceIdType.MESH,
            )
        pl.semaphore_wait(barrier_sem, 2)

        def hop_rdmas(h):
            cw = pltpu.make_async_remote_copy(
                src_ref=x_ref.at[pl.ds(0, half)] if h == 0 else cw_ref.at[h - 1],
                dst_ref=cw_ref.at[h],
                send_sem=send_cw.at[h],
                recv_sem=recv_cw.at[h],
                device_id=(right,),
                device_id_type=pl.DeviceIdType.MESH,
            )
            ccw = pltpu.make_async_remote_copy(
                src_ref=x_ref.at[pl.ds(half, half)] if h == 0 else ccw_ref.at[h - 1],
                dst_ref=ccw_ref.at[h],
                send_sem=send_ccw.at[h],
                recv_sem=recv_ccw.at[h],
                device_id=(left,),
                device_id_type=pl.DeviceIdType.MESH,
            )
            return cw, ccw

        sc = s_ref[0, 0]
        w = w_ref[...]

        rdmas = [hop_rdmas(0)]
        rdmas[0][0].start()
        rdmas[0][1].start()

        out_ref[pl.ds(my * m_per, m_per), :] = (
            jnp.dot(x_ref[...].astype(jnp.bfloat16), w,
                    preferred_element_type=jnp.float32) * sc
        )

        for h in range(N_HOP):
            cw, ccw = rdmas[h]
            cw.wait_recv()
            ccw.wait_recv()
            if h + 1 < N_HOP:
                nxt = hop_rdmas(h + 1)
                nxt[0].start()
                nxt[1].start()
                rdmas.append(nxt)
            o_cw = lax.rem(my + N_DEV - 1 - h, N_DEV)
            o_ccw = lax.rem(my + 1 + h, N_DEV)
            out_ref[pl.ds(o_cw * m_per, half), :] = (
                jnp.dot(cw_ref[h].astype(jnp.bfloat16), w,
                        preferred_element_type=jnp.float32) * sc
            )
            out_ref[pl.ds(o_ccw * m_per + half, half), :] = (
                jnp.dot(ccw_ref[h].astype(jnp.bfloat16), w,
                        preferred_element_type=jnp.float32) * sc
            )

        for cw, ccw in rdmas:
            cw.wait_send()
            ccw.wait_send()

    return pl.pallas_call(
        body,
        out_shape=jax.ShapeDtypeStruct((N_DEV * m_per, n_per), jnp.float32),
        in_specs=[
            pl.BlockSpec(memory_space=pltpu.SMEM),
            pl.BlockSpec(memory_space=pltpu.VMEM),
            pl.BlockSpec(memory_space=pltpu.VMEM),
        ],
        out_specs=pl.BlockSpec(memory_space=pltpu.VMEM),
        scratch_shapes=[
            pltpu.VMEM((N_HOP, half, k), jnp.float8_e4m3fn),
            pltpu.VMEM((N_HOP, half, k), jnp.float8_e4m3fn),
            pltpu.SemaphoreType.DMA((N_HOP,)),
            pltpu.SemaphoreType.DMA((N_HOP,)),
            pltpu.SemaphoreType.DMA((N_HOP,)),
            pltpu.SemaphoreType.DMA((N_HOP,)),
        ],
        compiler_params=pltpu.CompilerParams(collective_id=0),
    )(s, x8, w16)


# device time: 93648 ns/iter; 2.1321x vs baseline; 1.1341x over previous
import jax
import jax.numpy as jnp
from jax import lax
from jax.experimental import pallas as pl
from jax.experimental.pallas import tpu as pltpu

N_DEV = 8
N_HOP = N_DEV - 1


def kernel(x, w_mat, scale_x, scale_w):
    m_per, k = x.shape
    half = m_per // 2
    qtr = half // 2
    _, n_per = w_mat.shape

    x8 = x.astype(jnp.float8_e4m3fn)
    w16 = w_mat.astype(jnp.bfloat16)
    s = (scale_x * scale_w).reshape(1, 1)

    def body(s_ref, x_ref, w_ref, out_ref,
             cw_ref, ccw_ref, send_cw, recv_cw, send_ccw, recv_ccw):
        my = lax.axis_index("i")
        left = lax.rem(my + N_DEV - 1, N_DEV)
        right = lax.rem(my + 1, N_DEV)

        barrier_sem = pltpu.get_barrier_semaphore()
        for nbr in (left, right):
            pl.semaphore_signal(
                barrier_sem, inc=1,
                device_id=(nbr,), device_id_type=pl.DeviceIdType.MESH,
            )
        pl.semaphore_wait(barrier_sem, 2)

        def hop_rdmas(h, q):
            row = pl.ds(q * qtr, qtr)
            cw = pltpu.make_async_remote_copy(
                src_ref=(x_ref.at[pl.ds(q * qtr, qtr)] if h == 0
                         else cw_ref.at[h - 1, row]),
                dst_ref=cw_ref.at[h, row],
                send_sem=send_cw.at[h, q],
                recv_sem=recv_cw.at[h, q],
                device_id=(right,),
                device_id_type=pl.DeviceIdType.MESH,
            )
            ccw = pltpu.make_async_remote_copy(
                src_ref=(x_ref.at[pl.ds(half + q * qtr, qtr)] if h == 0
                         else ccw_ref.at[h - 1, row]),
                dst_ref=ccw_ref.at[h, row],
                send_sem=send_ccw.at[h, q],
                recv_sem=recv_ccw.at[h, q],
                device_id=(left,),
                device_id_type=pl.DeviceIdType.MESH,
            )
            return cw, ccw

        sc = s_ref[0, 0]
        w = w_ref[...]

        rdmas = {(0, q): hop_rdmas(0, q) for q in range(2)}
        for q in range(2):
            rdmas[0, q][0].start()
            rdmas[0, q][1].start()

        out_ref[pl.ds(my * m_per, m_per), :] = (
            jnp.dot(x_ref[...].astype(jnp.bfloat16), w,
                    preferred_element_type=jnp.float32) * sc
        )

        for h in range(N_HOP):
            for q in range(2):
                cw, ccw = rdmas[h, q]
                cw.wait_recv()
                ccw.wait_recv()
                if h + 1 < N_HOP:
                    nxt = hop_rdmas(h + 1, q)
                    nxt[0].start()
                    nxt[1].start()
                    rdmas[h + 1, q] = nxt
            o_cw = lax.rem(my + N_DEV - 1 - h, N_DEV)
            o_ccw = lax.rem(my + 1 + h, N_DEV)
            out_ref[pl.ds(o_cw * m_per, half), :] = (
                jnp.dot(cw_ref[h].astype(jnp.bfloat16), w,
                        preferred_element_type=jnp.float32) * sc
            )
            out_ref[pl.ds(o_ccw * m_per + half, half), :] = (
                jnp.dot(ccw_ref[h].astype(jnp.bfloat16), w,
                        preferred_element_type=jnp.float32) * sc
            )

        for cw, ccw in rdmas.values():
            cw.wait_send()
            ccw.wait_send()

    return pl.pallas_call(
        body,
        out_shape=jax.ShapeDtypeStruct((N_DEV * m_per, n_per), jnp.float32),
        in_specs=[
            pl.BlockSpec(memory_space=pltpu.SMEM),
            pl.BlockSpec(memory_space=pltpu.VMEM),
            pl.BlockSpec(memory_space=pltpu.VMEM),
        ],
        out_specs=pl.BlockSpec(memory_space=pltpu.VMEM),
        scratch_shapes=[
            pltpu.VMEM((N_HOP, half, k), jnp.float8_e4m3fn),
            pltpu.VMEM((N_HOP, half, k), jnp.float8_e4m3fn),
            pltpu.SemaphoreType.DMA((N_HOP, 2)),
            pltpu.SemaphoreType.DMA((N_HOP, 2)),
            pltpu.SemaphoreType.DMA((N_HOP, 2)),
            pltpu.SemaphoreType.DMA((N_HOP, 2)),
        ],
        compiler_params=pltpu.CompilerParams(collective_id=0),
    )(s, x8, w16)
